# baseline (speedup 1.0000x reference)
import functools

import jax
import jax.numpy as jnp
from jax import lax
from jax.experimental import pallas as pl
from jax.experimental.pallas import tpu as pltpu
from jax.experimental.pallas import tpu_sc as plsc

BATCH = 16384
EMBED = 32
VOCAB1 = 1000001
NC = 2
NS = 16
L = 16
NW = NC * NS
BPW = BATCH // NW


def _rsqrt16(x):
    i = plsc.bitcast(x, jnp.int32)
    i = jnp.int32(0x5F3759DF) - lax.shift_right_logical(i, 1)
    y = plsc.bitcast(i, jnp.float32)
    xh = x * jnp.float32(0.5)
    for _ in range(3):
        y = y * (jnp.float32(1.5) - xh * y * y)
    return y


def _make_kernel():
    mesh = plsc.VectorSubcoreMesh(
        core_axis_name="c", subcore_axis_name="s",
        num_cores=NC, num_subcores=NS)

    @functools.partial(
        pl.kernel,
        out_type=jax.ShapeDtypeStruct((EMBED, BATCH), jnp.float32),
        mesh=mesh,
        scratch_types=[
            pltpu.VMEM((BPW,), jnp.int32),
            pltpu.VMEM((2, BPW), jnp.int32),
            pltpu.VMEM((EMBED, BPW), jnp.float32),
            pltpu.SemaphoreType.DMA,
            pltpu.SemaphoreType.DMA,
        ],
        compiler_params=pltpu.CompilerParams(
            needs_layout_passes=False, use_tc_tiling_on_sc=False),
    )
    def sc_embed_norm(idx_hbm, flat_hbm, out_hbm,
                      wbase_v, widx_v, cols_v, sem0, sem1):
        wid = lax.axis_index("s") * NC + lax.axis_index("c")
        base = wid * BPW
        sems = (sem0, sem1)

        pltpu.sync_copy(idx_hbm.at[pl.ds(base, BPW)], wbase_v)
        for k in range(BPW // L):
            sl = pl.ds(k * L, L)
            wbase_v[sl] = lax.shift_left(wbase_v[sl] + 1, 5)

        def start(slot, d):
            for k in range(BPW // L):
                sl = pl.ds(k * L, L)
                widx_v[slot, sl] = wbase_v[sl] + jnp.int32(d)
            return pltpu.async_copy(
                flat_hbm.at[widx_v.at[slot]], cols_v.at[d], sems[slot])

        cp = start(0, 0)
        for d in range(EMBED):
            if d + 1 < EMBED:
                nxt = start((d + 1) & 1, d + 1)
            cp.wait()
            if d + 1 < EMBED:
                cp = nxt

        for k in range(BPW // L):
            sl = pl.ds(k * L, L)
            s2 = jnp.zeros((L,), jnp.float32)
            vals = []
            for d in range(EMBED):
                v = cols_v[d, sl]
                vals.append(v)
                s2 = s2 + v * v
            rr = _rsqrt16(jnp.maximum(s2, jnp.float32(1e-12)))
            for d in range(EMBED):
                cols_v[d, sl] = vals[d] * rr

        pltpu.sync_copy(cols_v, out_hbm.at[:, pl.ds(base, BPW)])

    return sc_embed_norm


_KERNEL = _make_kernel()


def kernel(customer_id, embedding_table):
    idx = customer_id.reshape(-1)
    flat = embedding_table.reshape(-1)
    out_t = _KERNEL(idx, flat)
    return out_t.T
